# baseline (speedup 1.0000x reference)
import functools

import jax
import jax.numpy as jnp
from jax import lax
from jax.experimental import pallas as pl
from jax.experimental.pallas import tpu as pltpu
from jax.experimental.pallas import tpu_sc as plsc

N_NODES = 10000
NC, NS = 2, 16
NW = NC * NS
GROUP = 128
GROUPS_PER_TILE = 88
EDGES_PAD = NW * GROUPS_PER_TILE * GROUP
ACC_ROWS = 10112
ROWS_PER_TILE = ACC_ROWS // NS
CNT_W = 8


def _make_sc_segment_sum(feat_w, with_count):
    mesh = plsc.VectorSubcoreMesh(core_axis_name="c", subcore_axis_name="s",
                                  num_cores=NC, num_subcores=NS)

    out_type = [jax.ShapeDtypeStruct((NC, ACC_ROWS, feat_w), jnp.float32)]
    scratch = [
        pltpu.VMEM((GROUPS_PER_TILE, GROUP), jnp.int32),
        pltpu.VMEM((GROUPS_PER_TILE, GROUP), jnp.int32),
        pltpu.VMEM((GROUP, feat_w), jnp.float32),
        pltpu.VMEM((GROUP, feat_w), jnp.float32),
        pltpu.VMEM_SHARED((ACC_ROWS, feat_w), jnp.float32),
        pltpu.SemaphoreType.DMA,
        pltpu.SemaphoreType.DMA,
    ]
    if with_count:
        out_type.append(jax.ShapeDtypeStruct((NC, ACC_ROWS, CNT_W), jnp.float32))
        scratch += [
            pltpu.VMEM((GROUP, CNT_W), jnp.float32),
            pltpu.VMEM_SHARED((ACC_ROWS, CNT_W), jnp.float32),
        ]

    def body(h_hbm, src_hbm, dst_hbm, zero_hbm, *rest):
        if with_count:
            (zero8_hbm, one_hbm, out_hbm, cnt_hbm,
             src_v, dst_v, rows_a, rows_b, acc, sem_a, sem_b,
             ones_v, cnt_acc) = rest
        else:
            (out_hbm, src_v, dst_v, rows_a, rows_b, acc,
             sem_a, sem_b) = rest

        c = lax.axis_index("c")
        s = lax.axis_index("s")
        wid = c * NS + s
        row0 = s * ROWS_PER_TILE

        pltpu.sync_copy(zero_hbm, acc.at[pl.ds(row0, ROWS_PER_TILE)])
        if with_count:
            pltpu.sync_copy(zero8_hbm, cnt_acc.at[pl.ds(row0, ROWS_PER_TILE)])
            pltpu.sync_copy(one_hbm, ones_v)
        g0 = wid * GROUPS_PER_TILE
        pltpu.sync_copy(src_hbm.at[pl.ds(g0, GROUPS_PER_TILE)], src_v)
        pltpu.sync_copy(dst_hbm.at[pl.ds(g0, GROUPS_PER_TILE)], dst_v)
        plsc.subcore_barrier()

        pltpu.async_copy(h_hbm.at[src_v.at[0]], rows_a, sem_a).wait()

        def step(i, carry):
            g = 2 * i
            cp_b = pltpu.async_copy(h_hbm.at[src_v.at[g + 1]], rows_b, sem_b)
            pltpu.sync_copy(rows_a, acc.at[dst_v.at[g]], add=True)
            if with_count:
                pltpu.sync_copy(ones_v, cnt_acc.at[dst_v.at[g]], add=True)
            cp_b.wait()
            cp_a = pltpu.async_copy(h_hbm.at[src_v.at[g + 2]], rows_a, sem_a)
            pltpu.sync_copy(rows_b, acc.at[dst_v.at[g + 1]], add=True)
            if with_count:
                pltpu.sync_copy(ones_v, cnt_acc.at[dst_v.at[g + 1]], add=True)
            cp_a.wait()
            return carry

        lax.fori_loop(0, GROUPS_PER_TILE // 2 - 1, step, 0)
        g = GROUPS_PER_TILE - 2
        cp_b = pltpu.async_copy(h_hbm.at[src_v.at[g + 1]], rows_b, sem_b)
        pltpu.sync_copy(rows_a, acc.at[dst_v.at[g]], add=True)
        if with_count:
            pltpu.sync_copy(ones_v, cnt_acc.at[dst_v.at[g]], add=True)
        cp_b.wait()
        pltpu.sync_copy(rows_b, acc.at[dst_v.at[g + 1]], add=True)
        if with_count:
            pltpu.sync_copy(ones_v, cnt_acc.at[dst_v.at[g + 1]], add=True)

        plsc.subcore_barrier()
        pltpu.sync_copy(acc.at[pl.ds(row0, ROWS_PER_TILE)],
                        out_hbm.at[c, pl.ds(row0, ROWS_PER_TILE)])
        if with_count:
            pltpu.sync_copy(cnt_acc.at[pl.ds(row0, ROWS_PER_TILE)],
                            cnt_hbm.at[c, pl.ds(row0, ROWS_PER_TILE)])

    return pl.kernel(body, out_type=out_type, mesh=mesh, scratch_types=scratch,
                     compiler_params=pltpu.CompilerParams(
                         use_tc_tiling_on_sc=False),
                     name=f"sc_segsum_{feat_w}" + ("_cnt" if with_count else ""))


def _lin_body(x_ref, w_ref, b_ref, o_ref):
    o_ref[...] = jnp.dot(x_ref[...], w_ref[...],
                         preferred_element_type=jnp.float32,
                         precision=lax.Precision.HIGHEST) + b_ref[...]


def _linear(x, w, b):
    return pl.pallas_call(
        _lin_body,
        out_shape=jax.ShapeDtypeStruct((x.shape[0], w.shape[1]), jnp.float32),
    )(x, w, b.reshape(1, -1))


def _combine_body(h_ref, s_ref, cnt_ref, w1_ref, b1_ref, wn_ref, bn_ref, o_ref):
    hw = h_ref.shape[1]
    ssum = s_ref[0, :N_NODES, :] + s_ref[1, :N_NODES, :]
    cnt = cnt_ref[0, :N_NODES, 0:1] + cnt_ref[1, :N_NODES, 0:1]
    aggr = ssum / cnt
    w1 = w1_ref[...]
    dot = functools.partial(jnp.dot, preferred_element_type=jnp.float32,
                            precision=lax.Precision.HIGHEST)
    act = jnp.maximum(dot(h_ref[...], w1[:hw]) + dot(aggr, w1[hw:])
                      + b1_ref[...], 0.0)
    o_ref[...] = dot(act, wn_ref[...]) + bn_ref[...]


def _combine(h, s_part, cnt_part, w1, b1, w_next, b_next):
    return pl.pallas_call(
        _combine_body,
        out_shape=jax.ShapeDtypeStruct((N_NODES, w_next.shape[1]), jnp.float32),
    )(h, s_part, cnt_part, w1, b1.reshape(1, -1), w_next, b_next.reshape(1, -1))


def kernel(x, edge_index, c1_W2, c1_b2, c1_W1, c1_b1,
           c2_W2, c2_b2, c2_W1, c2_b1, lin_W, lin_b):
    n = x.shape[0]
    loop = jnp.arange(n, dtype=jnp.int32)
    pad = EDGES_PAD - edge_index.shape[1] - n
    pad_dst = n + jnp.arange(pad, dtype=jnp.int32) % (ACC_ROWS - n)
    src = jnp.concatenate(
        [edge_index[0], loop, jnp.zeros((pad,), jnp.int32)]).reshape(-1, GROUP)
    dst = jnp.concatenate(
        [edge_index[1], loop, pad_dst]).reshape(-1, GROUP)

    zero64 = jnp.zeros((ROWS_PER_TILE, 64), jnp.float32)
    zero32 = jnp.zeros((ROWS_PER_TILE, 32), jnp.float32)
    zero8 = jnp.zeros((ROWS_PER_TILE, CNT_W), jnp.float32)
    ones = jnp.ones((GROUP, CNT_W), jnp.float32)

    h1 = _linear(x, c1_W2, c1_b2)
    s1, cnt = _make_sc_segment_sum(64, True)(
        h1, src, dst, zero64, zero8, ones)
    h2 = _combine(h1, s1, cnt, c1_W1, c1_b1, c2_W2, c2_b2)
    (s2,) = _make_sc_segment_sum(32, False)(h2, src, dst, zero32)
    return _combine(h2, s2, cnt, c2_W1, c2_b1, lin_W, lin_b)


# speedup vs baseline: 3.3904x; 3.3904x over previous
import functools

import jax
import jax.numpy as jnp
from jax import lax
from jax.experimental import pallas as pl
from jax.experimental.pallas import tpu as pltpu
from jax.experimental.pallas import tpu_sc as plsc

N_NODES = 10000
NC, NS = 2, 16
NW = NC * NS
GROUP = 128
GROUPS_PER_TILE = 88
EDGES_PAD = NW * GROUPS_PER_TILE * GROUP
ACC_ROWS = 10112
ROWS_PER_TILE = ACC_ROWS // NS
CNT_W = 8


def _make_sc_segment_sum(feat_w, with_count):
    mesh = plsc.VectorSubcoreMesh(core_axis_name="c", subcore_axis_name="s",
                                  num_cores=NC, num_subcores=NS)

    out_type = [jax.ShapeDtypeStruct((NC, ACC_ROWS, feat_w), jnp.float32)]
    scratch = [
        pltpu.VMEM((GROUPS_PER_TILE, GROUP), jnp.int32),
        pltpu.VMEM((GROUPS_PER_TILE, GROUP), jnp.int32),
        pltpu.VMEM((GROUP, feat_w), jnp.float32),
        pltpu.VMEM((GROUP, feat_w), jnp.float32),
        pltpu.VMEM_SHARED((ACC_ROWS, feat_w), jnp.float32),
        pltpu.SemaphoreType.DMA,
        pltpu.SemaphoreType.DMA,
    ]
    if with_count:
        out_type.append(jax.ShapeDtypeStruct((NC, ACC_ROWS, CNT_W), jnp.float32))
        scratch += [
            pltpu.VMEM((GROUP, CNT_W), jnp.float32),
            pltpu.VMEM_SHARED((ACC_ROWS, CNT_W), jnp.float32),
        ]

    def body(h_hbm, src_hbm, dst_hbm, zero_hbm, *rest):
        if with_count:
            (zero8_hbm, one_hbm, out_hbm, cnt_hbm,
             src_v, dst_v, rows_a, rows_b, acc, sem_a, sem_b,
             ones_v, cnt_acc) = rest
        else:
            (out_hbm, src_v, dst_v, rows_a, rows_b, acc,
             sem_a, sem_b) = rest

        c = lax.axis_index("c")
        s = lax.axis_index("s")
        wid = c * NS + s
        row0 = s * ROWS_PER_TILE

        pltpu.sync_copy(zero_hbm, acc.at[pl.ds(row0, ROWS_PER_TILE)])
        if with_count:
            pltpu.sync_copy(zero8_hbm, cnt_acc.at[pl.ds(row0, ROWS_PER_TILE)])
            pltpu.sync_copy(one_hbm, ones_v)
        g0 = wid * GROUPS_PER_TILE
        pltpu.sync_copy(src_hbm.at[pl.ds(g0, GROUPS_PER_TILE)], src_v)
        pltpu.sync_copy(dst_hbm.at[pl.ds(g0, GROUPS_PER_TILE)], dst_v)
        plsc.subcore_barrier()

        pltpu.async_copy(h_hbm.at[src_v.at[0]], rows_a, sem_a).wait()

        def step(i, carry):
            g = 2 * i
            cp_b = pltpu.async_copy(h_hbm.at[src_v.at[g + 1]], rows_b, sem_b)
            pltpu.sync_copy(rows_a, acc.at[dst_v.at[g]], add=True)
            if with_count:
                pltpu.sync_copy(ones_v, cnt_acc.at[dst_v.at[g]], add=True)
            cp_b.wait()
            cp_a = pltpu.async_copy(h_hbm.at[src_v.at[g + 2]], rows_a, sem_a)
            pltpu.sync_copy(rows_b, acc.at[dst_v.at[g + 1]], add=True)
            if with_count:
                pltpu.sync_copy(ones_v, cnt_acc.at[dst_v.at[g + 1]], add=True)
            cp_a.wait()
            return carry

        lax.fori_loop(0, GROUPS_PER_TILE // 2 - 1, step, 0)
        g = GROUPS_PER_TILE - 2
        cp_b = pltpu.async_copy(h_hbm.at[src_v.at[g + 1]], rows_b, sem_b)
        pltpu.sync_copy(rows_a, acc.at[dst_v.at[g]], add=True)
        if with_count:
            pltpu.sync_copy(ones_v, cnt_acc.at[dst_v.at[g]], add=True)
        cp_b.wait()
        pltpu.sync_copy(rows_b, acc.at[dst_v.at[g + 1]], add=True)
        if with_count:
            pltpu.sync_copy(ones_v, cnt_acc.at[dst_v.at[g + 1]], add=True)

        plsc.subcore_barrier()
        pltpu.sync_copy(acc.at[pl.ds(row0, ROWS_PER_TILE)],
                        out_hbm.at[c, pl.ds(row0, ROWS_PER_TILE)])
        if with_count:
            pltpu.sync_copy(cnt_acc.at[pl.ds(row0, ROWS_PER_TILE)],
                            cnt_hbm.at[c, pl.ds(row0, ROWS_PER_TILE)])

    return pl.kernel(body, out_type=out_type, mesh=mesh, scratch_types=scratch,
                     compiler_params=pltpu.CompilerParams(
                         use_tc_tiling_on_sc=False),
                     name=f"sc_segsum_{feat_w}" + ("_cnt" if with_count else ""))


def _lin_body(x_ref, w_ref, b_ref, o_ref):
    o_ref[...] = jnp.dot(x_ref[...], w_ref[...],
                         preferred_element_type=jnp.float32,
                         precision=lax.Precision.HIGHEST) + b_ref[...]


def _linear(x, w, b):
    return pl.pallas_call(
        _lin_body,
        out_shape=jax.ShapeDtypeStruct((x.shape[0], w.shape[1]), jnp.float32),
    )(x, w, b.reshape(1, -1))


def _combine_body(h_ref, s_ref, cnt_ref, w1_ref, b1_ref, wn_ref, bn_ref, o_ref):
    hw = h_ref.shape[1]
    ssum = s_ref[0, :N_NODES, :] + s_ref[1, :N_NODES, :]
    cnt = cnt_ref[0, :N_NODES, 0:1] + cnt_ref[1, :N_NODES, 0:1]
    aggr = ssum / cnt
    w1 = w1_ref[...]
    dot = functools.partial(jnp.dot, preferred_element_type=jnp.float32,
                            precision=lax.Precision.HIGHEST)
    act = jnp.maximum(dot(h_ref[...], w1[:hw]) + dot(aggr, w1[hw:])
                      + b1_ref[...], 0.0)
    o_ref[...] = dot(act, wn_ref[...]) + bn_ref[...]


def _combine(h, s_part, cnt_part, w1, b1, w_next, b_next):
    return pl.pallas_call(
        _combine_body,
        out_shape=jax.ShapeDtypeStruct((N_NODES, w_next.shape[1]), jnp.float32),
    )(h, s_part, cnt_part, w1, b1.reshape(1, -1), w_next, b_next.reshape(1, -1))


def kernel(x, edge_index, c1_W2, c1_b2, c1_W1, c1_b1,
           c2_W2, c2_b2, c2_W1, c2_b1, lin_W, lin_b):
    n = x.shape[0]
    loop = jnp.arange(n, dtype=jnp.int32)
    pad = EDGES_PAD - edge_index.shape[1] - n
    pad_iota = jnp.arange(pad, dtype=jnp.int32)
    pad_dst = n + pad_iota % (ACC_ROWS - n)
    src = jnp.concatenate(
        [edge_index[0], loop, pad_iota % n]).reshape(-1, GROUP)
    dst = jnp.concatenate(
        [edge_index[1], loop, pad_dst]).reshape(-1, GROUP)

    zero64 = jnp.zeros((ROWS_PER_TILE, 64), jnp.float32)
    zero32 = jnp.zeros((ROWS_PER_TILE, 32), jnp.float32)
    zero8 = jnp.zeros((ROWS_PER_TILE, CNT_W), jnp.float32)
    ones = jnp.ones((GROUP, CNT_W), jnp.float32)

    h1 = _linear(x, c1_W2, c1_b2)
    s1, cnt = _make_sc_segment_sum(64, True)(
        h1, src, dst, zero64, zero8, ones)
    h2 = _combine(h1, s1, cnt, c1_W1, c1_b1, c2_W2, c2_b2)
    (s2,) = _make_sc_segment_sum(32, False)(h2, src, dst, zero32)
    return _combine(h2, s2, cnt, c2_W1, c2_b1, lin_W, lin_b)


# speedup vs baseline: 3.6679x; 1.0819x over previous
import functools

import jax
import jax.numpy as jnp
from jax import lax
from jax.experimental import pallas as pl
from jax.experimental.pallas import tpu as pltpu
from jax.experimental.pallas import tpu_sc as plsc

N_NODES = 10000
NC, NS = 2, 16
NW = NC * NS
GROUP = 128
GROUPS_PER_TILE = 80
EDGES_PAD = NW * GROUPS_PER_TILE * GROUP
ACC_ROWS = 10112
ROWS_PER_TILE = ACC_ROWS // NS
AUG = 8


def _sc_segment_sum(feat_w):
    mesh = plsc.VectorSubcoreMesh(core_axis_name="c", subcore_axis_name="s",
                                  num_cores=NC, num_subcores=NS)
    out_type = jax.ShapeDtypeStruct((NC, ACC_ROWS, feat_w), jnp.float32)
    scratch = [
        pltpu.VMEM((GROUPS_PER_TILE, GROUP), jnp.int32),
        pltpu.VMEM((GROUPS_PER_TILE, GROUP), jnp.int32),
        pltpu.VMEM((GROUP, feat_w), jnp.float32),
        pltpu.VMEM((GROUP, feat_w), jnp.float32),
        pltpu.VMEM_SHARED((ACC_ROWS, feat_w), jnp.float32),
        pltpu.SemaphoreType.DMA,
        pltpu.SemaphoreType.DMA,
    ]

    def body(h_hbm, src_hbm, dst_hbm, zero_hbm, out_hbm,
             src_v, dst_v, rows_a, rows_b, acc, sem_a, sem_b):
        c = lax.axis_index("c")
        s = lax.axis_index("s")
        wid = c * NS + s
        row0 = s * ROWS_PER_TILE

        pltpu.sync_copy(zero_hbm, acc.at[pl.ds(row0, ROWS_PER_TILE)])
        g0 = wid * GROUPS_PER_TILE
        pltpu.sync_copy(src_hbm.at[pl.ds(g0, GROUPS_PER_TILE)], src_v)
        pltpu.sync_copy(dst_hbm.at[pl.ds(g0, GROUPS_PER_TILE)], dst_v)
        plsc.subcore_barrier()

        pltpu.async_copy(h_hbm.at[src_v.at[0]], rows_a, sem_a).wait()

        def step(i, carry):
            g = 2 * i
            cp_b = pltpu.async_copy(h_hbm.at[src_v.at[g + 1]], rows_b, sem_b)
            pltpu.sync_copy(rows_a, acc.at[dst_v.at[g]], add=True)
            cp_b.wait()
            cp_a = pltpu.async_copy(h_hbm.at[src_v.at[g + 2]], rows_a, sem_a)
            pltpu.sync_copy(rows_b, acc.at[dst_v.at[g + 1]], add=True)
            cp_a.wait()
            return carry

        lax.fori_loop(0, GROUPS_PER_TILE // 2 - 1, step, 0)
        g = GROUPS_PER_TILE - 2
        cp_b = pltpu.async_copy(h_hbm.at[src_v.at[g + 1]], rows_b, sem_b)
        pltpu.sync_copy(rows_a, acc.at[dst_v.at[g]], add=True)
        cp_b.wait()
        pltpu.sync_copy(rows_b, acc.at[dst_v.at[g + 1]], add=True)

        plsc.subcore_barrier()
        pltpu.sync_copy(acc.at[pl.ds(row0, ROWS_PER_TILE)],
                        out_hbm.at[c, pl.ds(row0, ROWS_PER_TILE)])

    return pl.kernel(body, out_type=out_type, mesh=mesh, scratch_types=scratch,
                     compiler_params=pltpu.CompilerParams(
                         use_tc_tiling_on_sc=False),
                     name=f"sc_segsum_{feat_w}")


_DOT = functools.partial(jnp.dot, preferred_element_type=jnp.float32,
                         precision=lax.Precision.HIGHEST)


def _lin1_body(x_ref, w_ref, b_ref, o_ref):
    h = _DOT(x_ref[...], w_ref[...]) + b_ref[...]
    o_ref[...] = jnp.concatenate(
        [h, jnp.ones((h.shape[0], AUG), jnp.float32)], axis=1)


def _lin1(x, w, b):
    return pl.pallas_call(
        _lin1_body,
        out_shape=jax.ShapeDtypeStruct((x.shape[0], w.shape[1] + AUG),
                                       jnp.float32),
    )(x, w, b.reshape(1, -1))


def _combine1_body(ha_ref, s_ref, w1_ref, b1_ref, w2_ref, b2_ref,
                   o_ref, inv_ref):
    h = ha_ref[:, :64]
    ssum = s_ref[0, :N_NODES, :] + s_ref[1, :N_NODES, :]
    inv = 1.0 / (ssum[:, 64:65] + 1.0)
    aggr = (ssum[:, :64] + h) * inv
    w1 = w1_ref[...]
    x1 = jnp.maximum(_DOT(h, w1[:64]) + _DOT(aggr, w1[64:]) + b1_ref[...], 0.0)
    o_ref[...] = _DOT(x1, w2_ref[...]) + b2_ref[...]
    inv_ref[...] = jnp.broadcast_to(inv, (N_NODES, AUG))


def _combine1(h_aug, s_part, w1, b1, w2, b2):
    return pl.pallas_call(
        _combine1_body,
        out_shape=[jax.ShapeDtypeStruct((N_NODES, w2.shape[1]), jnp.float32),
                   jax.ShapeDtypeStruct((N_NODES, AUG), jnp.float32)],
    )(h_aug, s_part, w1, b1.reshape(1, -1), w2, b2.reshape(1, -1))


def _combine2_body(h_ref, s_ref, inv_ref, w1_ref, b1_ref, w2_ref, b2_ref,
                   o_ref):
    h = h_ref[...]
    ssum = s_ref[0, :N_NODES, :] + s_ref[1, :N_NODES, :]
    aggr = (ssum + h) * inv_ref[:, 0:1]
    w1 = w1_ref[...]
    x2 = jnp.maximum(_DOT(h, w1[:32]) + _DOT(aggr, w1[32:]) + b1_ref[...], 0.0)
    o_ref[...] = _DOT(x2, w2_ref[...]) + b2_ref[...]


def _combine2(h, s_part, cnt_inv, w1, b1, w2, b2):
    return pl.pallas_call(
        _combine2_body,
        out_shape=jax.ShapeDtypeStruct((N_NODES, w2.shape[1]), jnp.float32),
    )(h, s_part, cnt_inv, w1, b1.reshape(1, -1), w2, b2.reshape(1, -1))


def kernel(x, edge_index, c1_W2, c1_b2, c1_W1, c1_b1,
           c2_W2, c2_b2, c2_W1, c2_b1, lin_W, lin_b):
    n = x.shape[0]
    pad = EDGES_PAD - edge_index.shape[1]
    pad_iota = jnp.arange(pad, dtype=jnp.int32)
    src = jnp.concatenate([edge_index[0], pad_iota % n]).reshape(-1, GROUP)
    dst = jnp.concatenate(
        [edge_index[1], n + pad_iota % (ACC_ROWS - n)]).reshape(-1, GROUP)

    zero72 = jnp.zeros((ROWS_PER_TILE, 64 + AUG), jnp.float32)
    zero32 = jnp.zeros((ROWS_PER_TILE, 32), jnp.float32)

    h1 = _lin1(x, c1_W2, c1_b2)
    s1 = _sc_segment_sum(64 + AUG)(h1, src, dst, zero72)
    h2, cnt_inv = _combine1(h1, s1, c1_W1, c1_b1, c2_W2, c2_b2)
    s2 = _sc_segment_sum(32)(h2, src, dst, zero32)
    return _combine2(h2, s2, cnt_inv, c2_W1, c2_b1, lin_W, lin_b)


# speedup vs baseline: 4.1949x; 1.1437x over previous
import functools

import jax
import jax.numpy as jnp
from jax import lax
from jax.experimental import pallas as pl
from jax.experimental.pallas import tpu as pltpu
from jax.experimental.pallas import tpu_sc as plsc

N_NODES = 10000
NC, NS = 2, 16
NW = NC * NS
GROUP = 128
GROUPS_PER_TILE = 80
EDGES_PAD = NW * GROUPS_PER_TILE * GROUP
ACC_ROWS = 10112
ROWS_PER_TILE = ACC_ROWS // NS
AUG = 8


def _sc_segment_sum(feat_w):
    mesh = plsc.VectorSubcoreMesh(core_axis_name="c", subcore_axis_name="s",
                                  num_cores=NC, num_subcores=NS)
    out_type = jax.ShapeDtypeStruct((NC, ACC_ROWS, feat_w), jnp.float32)
    scratch = [
        pltpu.VMEM((GROUPS_PER_TILE, GROUP), jnp.int32),
        pltpu.VMEM((GROUPS_PER_TILE, GROUP), jnp.int32),
        pltpu.VMEM((GROUP, feat_w), jnp.float32),
        pltpu.VMEM((GROUP, feat_w), jnp.float32),
        pltpu.VMEM_SHARED((ACC_ROWS, feat_w), jnp.float32),
        pltpu.SemaphoreType.DMA,
        pltpu.SemaphoreType.DMA,
    ]

    def body(h_hbm, src_hbm, dst_hbm, zero_hbm, out_hbm,
             src_v, dst_v, rows_a, rows_b, acc, sem_a, sem_b):
        c = lax.axis_index("c")
        s = lax.axis_index("s")
        wid = c * NS + s
        row0 = s * ROWS_PER_TILE

        pltpu.sync_copy(zero_hbm, acc.at[pl.ds(row0, ROWS_PER_TILE)])
        g0 = wid * GROUPS_PER_TILE
        pltpu.sync_copy(src_hbm.at[pl.ds(g0, GROUPS_PER_TILE)], src_v)
        pltpu.sync_copy(dst_hbm.at[pl.ds(g0, GROUPS_PER_TILE)], dst_v)
        plsc.subcore_barrier()

        pltpu.async_copy(h_hbm.at[src_v.at[0]], rows_a, sem_a).wait()

        def step(i, carry):
            g = 2 * i
            cp_b = pltpu.async_copy(h_hbm.at[src_v.at[g + 1]], rows_b, sem_b)
            pltpu.sync_copy(rows_a, acc.at[dst_v.at[g]], add=True)
            cp_b.wait()
            cp_a = pltpu.async_copy(h_hbm.at[src_v.at[g + 2]], rows_a, sem_a)
            pltpu.sync_copy(rows_b, acc.at[dst_v.at[g + 1]], add=True)
            cp_a.wait()
            return carry

        lax.fori_loop(0, GROUPS_PER_TILE // 2 - 1, step, 0)
        g = GROUPS_PER_TILE - 2
        cp_b = pltpu.async_copy(h_hbm.at[src_v.at[g + 1]], rows_b, sem_b)
        pltpu.sync_copy(rows_a, acc.at[dst_v.at[g]], add=True)
        cp_b.wait()
        pltpu.sync_copy(rows_b, acc.at[dst_v.at[g + 1]], add=True)

        plsc.subcore_barrier()
        pltpu.sync_copy(acc.at[pl.ds(row0, ROWS_PER_TILE)],
                        out_hbm.at[c, pl.ds(row0, ROWS_PER_TILE)])

    return pl.kernel(body, out_type=out_type, mesh=mesh, scratch_types=scratch,
                     compiler_params=pltpu.CompilerParams(
                         use_tc_tiling_on_sc=False),
                     name=f"sc_segsum_{feat_w}")


_DOT = functools.partial(jnp.dot, preferred_element_type=jnp.float32,
                         precision=lax.Precision.DEFAULT)


def _lin1_body(x_ref, w_ref, b_ref, o_ref):
    h = _DOT(x_ref[...], w_ref[...]) + b_ref[...]
    o_ref[...] = jnp.concatenate(
        [h, jnp.ones((h.shape[0], AUG), jnp.float32)], axis=1)


def _lin1(x, w, b):
    return pl.pallas_call(
        _lin1_body,
        out_shape=jax.ShapeDtypeStruct((x.shape[0], w.shape[1] + AUG),
                                       jnp.float32),
    )(x, w, b.reshape(1, -1))


def _combine1_body(ha_ref, s_ref, w1_ref, b1_ref, w2_ref, b2_ref,
                   o_ref, inv_ref):
    h = ha_ref[:, :64]
    ssum = s_ref[0, :N_NODES, :] + s_ref[1, :N_NODES, :]
    inv = 1.0 / (ssum[:, 64:65] + 1.0)
    aggr = (ssum[:, :64] + h) * inv
    w1 = w1_ref[...]
    x1 = jnp.maximum(_DOT(h, w1[:64]) + _DOT(aggr, w1[64:]) + b1_ref[...], 0.0)
    o_ref[...] = _DOT(x1, w2_ref[...]) + b2_ref[...]
    inv_ref[...] = jnp.broadcast_to(inv, (N_NODES, AUG))


def _combine1(h_aug, s_part, w1, b1, w2, b2):
    return pl.pallas_call(
        _combine1_body,
        out_shape=[jax.ShapeDtypeStruct((N_NODES, w2.shape[1]), jnp.float32),
                   jax.ShapeDtypeStruct((N_NODES, AUG), jnp.float32)],
    )(h_aug, s_part, w1, b1.reshape(1, -1), w2, b2.reshape(1, -1))


def _combine2_body(h_ref, s_ref, inv_ref, w1_ref, b1_ref, w2_ref, b2_ref,
                   o_ref):
    h = h_ref[...]
    ssum = s_ref[0, :N_NODES, :] + s_ref[1, :N_NODES, :]
    aggr = (ssum + h) * inv_ref[:, 0:1]
    w1 = w1_ref[...]
    x2 = jnp.maximum(_DOT(h, w1[:32]) + _DOT(aggr, w1[32:]) + b1_ref[...], 0.0)
    o_ref[...] = _DOT(x2, w2_ref[...]) + b2_ref[...]


def _combine2(h, s_part, cnt_inv, w1, b1, w2, b2):
    return pl.pallas_call(
        _combine2_body,
        out_shape=jax.ShapeDtypeStruct((N_NODES, w2.shape[1]), jnp.float32),
    )(h, s_part, cnt_inv, w1, b1.reshape(1, -1), w2, b2.reshape(1, -1))


def kernel(x, edge_index, c1_W2, c1_b2, c1_W1, c1_b1,
           c2_W2, c2_b2, c2_W1, c2_b1, lin_W, lin_b):
    n = x.shape[0]
    pad = EDGES_PAD - edge_index.shape[1]
    pad_iota = jnp.arange(pad, dtype=jnp.int32)
    src = jnp.concatenate([edge_index[0], pad_iota % n]).reshape(-1, GROUP)
    dst = jnp.concatenate(
        [edge_index[1], n + pad_iota % (ACC_ROWS - n)]).reshape(-1, GROUP)

    zero72 = jnp.zeros((ROWS_PER_TILE, 64 + AUG), jnp.float32)
    zero32 = jnp.zeros((ROWS_PER_TILE, 32), jnp.float32)

    h1 = _lin1(x, c1_W2, c1_b2)
    s1 = _sc_segment_sum(64 + AUG)(h1, src, dst, zero72)
    h2, cnt_inv = _combine1(h1, s1, c1_W1, c1_b1, c2_W2, c2_b2)
    s2 = _sc_segment_sum(32)(h2, src, dst, zero32)
    return _combine2(h2, s2, cnt_inv, c2_W1, c2_b1, lin_W, lin_b)
